# baseline (speedup 1.0000x reference)
import jax
import jax.numpy as jnp
from jax import lax
from jax.experimental import pallas as pl
from jax.experimental.pallas import tpu as pltpu
from jax.experimental.pallas import tpu_sc as plsc
import dataclasses
import functools

_sc_params = pltpu.CompilerParams()
if "needs_layout_passes" in pltpu.CompilerParams.__dataclass_fields__:
    _sc_params = dataclasses.replace(_sc_params, needs_layout_passes=False)

N = 524288
D = 128
H = 21
TASK = 3

NC = 2
NS = 16
NW = NC * NS
CHUNK = N // NW
KG = 128
BT = 2048
NBLK = N // BT
BPW = CHUNK // BT
NSTEP = NW * BPW
CP = N // NSTEP
NSEM = 4

_mesh = plsc.VectorSubcoreMesh(core_axis_name="c", subcore_axis_name="s")


def _wid():
    return lax.axis_index("s") * NC + lax.axis_index("c")


@functools.partial(
    pl.kernel,
    out_type=[
        jax.ShapeDtypeStruct((N, D), jnp.float32),
        jax.ShapeDtypeStruct((NW, CHUNK), jnp.int32),
        jax.ShapeDtypeStruct((NW, 16), jnp.int32),
    ],
    mesh=_mesh,
    scratch_types=[
        pltpu.VMEM((CHUNK,), jnp.int32),
        pltpu.VMEM((CHUNK + KG,), jnp.int32),
        pltpu.VMEM((KG, D), jnp.float32),
        pltpu.VMEM((KG, D), jnp.float32),
        pltpu.VMEM((16,), jnp.int32),
        pltpu.SemaphoreType.DMA,
        pltpu.SemaphoreType.DMA,
        pltpu.SemaphoreType.DMA,
        pltpu.SemaphoreType.DMA,
    ],
    compiler_params=_sc_params,
)
def _sc_compact_gather(x1_hbm, x0_hbm, xg_hbm, idx_hbm, cnt_hbm,
                       x1_v, idx_v, rows_a, rows_b, cnt_v,
                       sem_ga, sem_gb, sem_wa, sem_wb):
    w = _wid()
    base = w * CHUNK
    pltpu.sync_copy(x1_hbm.at[pl.ds(base, CHUNK)], x1_v)

    lanes = lax.iota(jnp.int32, 16)

    @pl.loop(0, CHUNK // 16, init_carry=jnp.int32(0), unroll=4)
    def cnt(i, c):
        v = x1_v[pl.ds(i * 16, 16)]
        m = v == TASK
        gi = (base + i * 16) + lanes
        plsc.store_compressed(idx_v.at[pl.ds(c, 16)], gi, mask=m)
        return c + jnp.sum(m.astype(jnp.int32))

    safe = jnp.full((16,), base, jnp.int32)
    for p in range(KG // 16):
        idx_v[pl.ds(cnt + p * 16, 16)] = safe

    cnt_v[...] = jnp.full((16,), cnt, jnp.int32)
    pltpu.sync_copy(cnt_v, cnt_hbm.at[w])
    pltpu.sync_copy(idx_v.at[pl.ds(0, CHUNK)], idx_hbm.at[w])

    nch = (cnt + (KG - 1)) // KG

    def _start_gather(j, buf, sem):
        pltpu.make_async_copy(
            x0_hbm.at[idx_v.at[pl.ds(j * KG, KG)]], buf, sem
        ).start()

    def _wait_gather(buf, sem):
        pltpu.make_async_copy(x0_hbm.at[pl.ds(0, KG)], buf, sem).wait()

    def _start_write(j, buf, sem):
        pltpu.make_async_copy(
            buf, xg_hbm.at[pl.ds(base + j * KG, KG)], sem
        ).start()

    def _wait_write(buf, sem):
        pltpu.make_async_copy(buf, xg_hbm.at[pl.ds(0, KG)], sem).wait()

    @pl.when(nch > 0)
    def _():
        _start_gather(0, rows_a, sem_ga)

        @pl.loop(0, nch)
        def _(j):
            is_a = lax.rem(j, 2) == 0

            @pl.when(is_a)
            def _():
                _wait_gather(rows_a, sem_ga)

                @pl.when(j + 1 < nch)
                def _():
                    @pl.when(j >= 1)
                    def _():
                        _wait_write(rows_b, sem_wb)
                    _start_gather(j + 1, rows_b, sem_gb)
                _start_write(j, rows_a, sem_wa)

            @pl.when(jnp.logical_not(is_a))
            def _():
                _wait_gather(rows_b, sem_gb)

                @pl.when(j + 1 < nch)
                def _():
                    _wait_write(rows_a, sem_wa)
                    _start_gather(j + 1, rows_a, sem_ga)
                _start_write(j, rows_b, sem_wb)

        @pl.when(lax.rem(nch, 2) == 1)
        def _():
            _wait_write(rows_a, sem_wa)

            @pl.when(nch >= 2)
            def _():
                _wait_write(rows_b, sem_wb)

        @pl.when(lax.rem(nch, 2) == 0)
        def _():
            _wait_write(rows_b, sem_wb)
            _wait_write(rows_a, sem_wa)


def _mlp_body(cnt_ref, xg_ref, w1_ref, b1_ref, w2h_ref, b2_ref,
              x0_any, h_ref, xo_any, sems):
    w = pl.program_id(0)
    j = pl.program_id(1)
    t = w * BPW + j

    def _cp(tt, slot):
        return pltpu.make_async_copy(
            x0_any.at[pl.ds(tt * CP, CP)], xo_any.at[pl.ds(tt * CP, CP)],
            sems.at[slot],
        )

    for k in range(NSEM):
        @pl.when(lax.rem(t, NSEM) == k)
        def _():
            @pl.when(t >= NSEM)
            def _():
                _cp(t - NSEM, k).wait()
            _cp(t, k).start()

    @pl.when(t == NSTEP - 1)
    def _():
        for k in range(NSEM):
            _cp(NSTEP - NSEM + k, (NSTEP - NSEM + k) % NSEM).wait()

    @pl.when(j * BT < cnt_ref[w, 0])
    def _():
        x = xg_ref[...]
        z = lax.dot_general(
            w1_ref[...], x, (((0,), (1,)), ((), ())),
            preferred_element_type=jnp.float32,
        )
        z = z + b1_ref[...]
        e = lax.erf(z * 0.7071067811865476)
        h_ref[0] = jnp.sum((z + z * e) * w2h_ref[...], axis=0, keepdims=True) \
            + b2_ref[...]


def _xg_map(w, j, cnt_ref):
    nb = (cnt_ref[w, 0] + (BT - 1)) // BT
    jc = jnp.minimum(j, jnp.maximum(nb - 1, 0))
    return (w * BPW + jc, 0)


def _h_map(w, j, cnt_ref):
    b, _ = _xg_map(w, j, cnt_ref)
    return (b, 0, 0)


def _tc_mlp(cnt16, xg, W1, b1, W2, b2, x0):
    h, x0_out = pl.pallas_call(
        _mlp_body,
        grid_spec=pltpu.PrefetchScalarGridSpec(
            num_scalar_prefetch=1,
            grid=(NW, BPW),
            in_specs=[
                pl.BlockSpec((BT, D), _xg_map),
                pl.BlockSpec((D, H), lambda w, j, c: (0, 0)),
                pl.BlockSpec((H, 1), lambda w, j, c: (0, 0)),
                pl.BlockSpec((H, 1), lambda w, j, c: (0, 0)),
                pl.BlockSpec((1, 1), lambda w, j, c: (0, 0)),
                pl.BlockSpec(memory_space=pl.ANY),
            ],
            out_specs=[
                pl.BlockSpec((1, 1, BT), _h_map),
                pl.BlockSpec(memory_space=pl.ANY),
            ],
            scratch_shapes=[pltpu.SemaphoreType.DMA((NSEM,))],
        ),
        out_shape=[
            jax.ShapeDtypeStruct((NBLK, 1, BT), jnp.float32),
            jax.ShapeDtypeStruct((N, D), jnp.float32),
        ],
    )(cnt16, xg, W1, b1.reshape(H, 1), 0.5 * W2, b2.reshape(1, 1), x0)
    return h, x0_out


@functools.partial(
    pl.kernel,
    out_type=jax.ShapeDtypeStruct((N,), jnp.float32),
    mesh=_mesh,
    scratch_types=[
        pltpu.VMEM((CHUNK,), jnp.int32),
        pltpu.VMEM((CHUNK,), jnp.float32),
        pltpu.VMEM((CHUNK,), jnp.float32),
        pltpu.VMEM((16,), jnp.int32),
    ],
    compiler_params=_sc_params,
)
def _sc_scatter(idx_hbm, cnt_hbm, h_hbm, x2_hbm, out_hbm,
                idx_v, h_v, out_v, cnt_v):
    w = _wid()
    base = w * CHUNK
    pltpu.sync_copy(cnt_hbm.at[w], cnt_v)
    pltpu.sync_copy(idx_hbm.at[w], idx_v)
    pltpu.sync_copy(h_hbm.at[pl.ds(base, CHUNK)], h_v)
    pltpu.sync_copy(x2_hbm.at[pl.ds(base, CHUNK)], out_v)
    cnt = cnt_v[...][0]
    lanes = lax.iota(jnp.int32, 16)

    @pl.loop(0, (cnt + 15) // 16)
    def _(k):
        off = k * 16
        iv = idx_v[pl.ds(off, 16)] - base
        hv = h_v[pl.ds(off, 16)]
        m = (off + lanes) < cnt
        plsc.store_scatter(out_v, [iv], hv, mask=m)

    pltpu.sync_copy(out_v, out_hbm.at[pl.ds(base, CHUNK)])


def kernel(x0, x1, x2, W1, b1, W2, b2):
    x1i = x1.astype(jnp.int32).reshape(N)
    xg, idxs, cnt16 = _sc_compact_gather(x1i, x0)
    h, x0_out = _tc_mlp(cnt16, xg, W1, b1, W2, b2, x0)
    x2_new = _sc_scatter(idxs, cnt16, h.reshape(N), x2.reshape(N))
    return (x0_out, x1, x2_new.reshape(N, 1))


# speedup vs baseline: 26.7656x; 26.7656x over previous
import jax
import jax.numpy as jnp
from jax import lax
from jax.experimental import pallas as pl
from jax.experimental.pallas import tpu as pltpu
from jax.experimental.pallas import tpu_sc as plsc
import dataclasses
import functools

_sc_params = pltpu.CompilerParams()
if "needs_layout_passes" in pltpu.CompilerParams.__dataclass_fields__:
    _sc_params = dataclasses.replace(_sc_params, needs_layout_passes=False)

N = 524288
D = 128
H = 21
TASK = 3

NC = 2
NS = 16
NW = NC * NS
CHUNK = N // NW
KG = 128
BT = 2048
NBLK = N // BT
BPW = CHUNK // BT
BC = 8192

_mesh = plsc.VectorSubcoreMesh(core_axis_name="c", subcore_axis_name="s")


def _wid():
    return lax.axis_index("s") * NC + lax.axis_index("c")


@functools.partial(
    pl.kernel,
    out_type=[
        jax.ShapeDtypeStruct((N, D), jnp.float32),
        jax.ShapeDtypeStruct((NW, CHUNK), jnp.int32),
        jax.ShapeDtypeStruct((NW, 16), jnp.int32),
    ],
    mesh=_mesh,
    scratch_types=[
        pltpu.VMEM((CHUNK,), jnp.int32),
        pltpu.VMEM((CHUNK + KG,), jnp.int32),
        pltpu.VMEM((KG, D), jnp.float32),
        pltpu.VMEM((KG, D), jnp.float32),
        pltpu.VMEM((16,), jnp.int32),
        pltpu.SemaphoreType.DMA,
        pltpu.SemaphoreType.DMA,
        pltpu.SemaphoreType.DMA,
        pltpu.SemaphoreType.DMA,
    ],
    compiler_params=_sc_params,
)
def _sc_compact_gather(x1_hbm, x0_hbm, xg_hbm, idx_hbm, cnt_hbm,
                       x1_v, idx_v, rows_a, rows_b, cnt_v,
                       sem_ga, sem_gb, sem_wa, sem_wb):
    w = _wid()
    base = w * CHUNK
    pltpu.sync_copy(x1_hbm.at[pl.ds(base, CHUNK)], x1_v)

    lanes = lax.iota(jnp.int32, 16)

    @pl.loop(0, CHUNK // 16, init_carry=jnp.int32(0), unroll=4)
    def cnt(i, c):
        v = x1_v[pl.ds(i * 16, 16)]
        m = v == TASK
        gi = (base + i * 16) + lanes
        plsc.store_compressed(idx_v.at[pl.ds(c, 16)], gi, mask=m)
        return c + jnp.sum(m.astype(jnp.int32))

    safe = jnp.full((16,), base, jnp.int32)
    for p in range(KG // 16):
        idx_v[pl.ds(cnt + p * 16, 16)] = safe

    cnt_v[...] = jnp.full((16,), cnt, jnp.int32)
    pltpu.sync_copy(cnt_v, cnt_hbm.at[w])
    pltpu.sync_copy(idx_v.at[pl.ds(0, CHUNK)], idx_hbm.at[w])

    nch = (cnt + (KG - 1)) // KG

    def _start_gather(j, buf, sem):
        pltpu.make_async_copy(
            x0_hbm.at[idx_v.at[pl.ds(j * KG, KG)]], buf, sem
        ).start()

    def _wait_gather(buf, sem):
        pltpu.make_async_copy(x0_hbm.at[pl.ds(0, KG)], buf, sem).wait()

    def _start_write(j, buf, sem):
        pltpu.make_async_copy(
            buf, xg_hbm.at[pl.ds(base + j * KG, KG)], sem
        ).start()

    def _wait_write(buf, sem):
        pltpu.make_async_copy(buf, xg_hbm.at[pl.ds(0, KG)], sem).wait()

    @pl.when(nch > 0)
    def _():
        _start_gather(0, rows_a, sem_ga)

        @pl.loop(0, nch)
        def _(j):
            is_a = lax.rem(j, 2) == 0

            @pl.when(is_a)
            def _():
                _wait_gather(rows_a, sem_ga)

                @pl.when(j + 1 < nch)
                def _():
                    @pl.when(j >= 1)
                    def _():
                        _wait_write(rows_b, sem_wb)
                    _start_gather(j + 1, rows_b, sem_gb)
                _start_write(j, rows_a, sem_wa)

            @pl.when(jnp.logical_not(is_a))
            def _():
                _wait_gather(rows_b, sem_gb)

                @pl.when(j + 1 < nch)
                def _():
                    _wait_write(rows_a, sem_wa)
                    _start_gather(j + 1, rows_a, sem_ga)
                _start_write(j, rows_b, sem_wb)

        @pl.when(lax.rem(nch, 2) == 1)
        def _():
            _wait_write(rows_a, sem_wa)

            @pl.when(nch >= 2)
            def _():
                _wait_write(rows_b, sem_wb)

        @pl.when(lax.rem(nch, 2) == 0)
        def _():
            _wait_write(rows_b, sem_wb)
            _wait_write(rows_a, sem_wa)


def _copy_body(x_ref, o_ref):
    o_ref[...] = x_ref[...]


def _tc_copy(x0):
    return pl.pallas_call(
        _copy_body,
        grid=(N // BC,),
        in_specs=[pl.BlockSpec((BC, D), lambda i: (i, 0))],
        out_specs=pl.BlockSpec((BC, D), lambda i: (i, 0)),
        out_shape=jax.ShapeDtypeStruct((N, D), jnp.float32),
    )(x0)


def _mlp_body(cnt_ref, xg_ref, wp_ref, xo_any, h_ref):
    w = pl.program_id(0)
    j = pl.program_id(1)

    @pl.when(j * BT < cnt_ref[w, 0])
    def _():
        x = xg_ref[...]
        z = lax.dot_general(
            wp_ref[0:D, :], x, (((0,), (1,)), ((), ())),
            preferred_element_type=jnp.float32,
        )
        z = z + wp_ref[D:D + 1, :].reshape(H, 1)
        e = lax.erf(z * 0.7071067811865476)
        w2h = wp_ref[D + 1:D + 2, :].reshape(H, 1)
        b2 = wp_ref[D + 2, 0]
        h_ref[0] = jnp.sum((z + z * e) * w2h, axis=0, keepdims=True) + b2


def _xg_map(w, j, cnt_ref):
    nb = (cnt_ref[w, 0] + (BT - 1)) // BT
    jc = jnp.minimum(j, jnp.maximum(nb - 1, 0))
    return (w * BPW + jc, 0)


def _h_map(w, j, cnt_ref):
    b, _ = _xg_map(w, j, cnt_ref)
    return (b, 0, 0)


def _tc_mlp(cnt16, xg, wpack, x0_out):
    return pl.pallas_call(
        _mlp_body,
        grid_spec=pltpu.PrefetchScalarGridSpec(
            num_scalar_prefetch=1,
            grid=(NW, BPW),
            in_specs=[
                pl.BlockSpec((BT, D), _xg_map),
                pl.BlockSpec((D + 3, H), lambda w, j, c: (0, 0)),
                pl.BlockSpec(memory_space=pl.ANY),
            ],
            out_specs=pl.BlockSpec((1, 1, BT), _h_map),
        ),
        out_shape=jax.ShapeDtypeStruct((NBLK, 1, BT), jnp.float32),
    )(cnt16, xg, wpack, x0_out)


@functools.partial(
    pl.kernel,
    out_type=jax.ShapeDtypeStruct((N,), jnp.float32),
    mesh=_mesh,
    scratch_types=[
        pltpu.VMEM((CHUNK,), jnp.int32),
        pltpu.VMEM((CHUNK,), jnp.float32),
        pltpu.VMEM((CHUNK,), jnp.float32),
        pltpu.VMEM((16,), jnp.int32),
    ],
    compiler_params=_sc_params,
)
def _sc_scatter(idx_hbm, cnt_hbm, h_hbm, x2_hbm, out_hbm,
                idx_v, h_v, out_v, cnt_v):
    w = _wid()
    base = w * CHUNK
    pltpu.sync_copy(cnt_hbm.at[w], cnt_v)
    pltpu.sync_copy(idx_hbm.at[w], idx_v)
    pltpu.sync_copy(h_hbm.at[pl.ds(base, CHUNK)], h_v)
    pltpu.sync_copy(x2_hbm.at[pl.ds(base, CHUNK)], out_v)
    cnt = cnt_v[...][0]
    lanes = lax.iota(jnp.int32, 16)

    @pl.loop(0, (cnt + 15) // 16)
    def _(k):
        off = k * 16
        iv = idx_v[pl.ds(off, 16)] - base
        hv = h_v[pl.ds(off, 16)]
        m = (off + lanes) < cnt
        plsc.store_scatter(out_v, [iv], hv, mask=m)

    pltpu.sync_copy(out_v, out_hbm.at[pl.ds(base, CHUNK)])


def kernel(x0, x1, x2, W1, b1, W2, b2):
    x1i = x1.astype(jnp.int32).reshape(N)
    wpack = jnp.concatenate(
        [W1, b1.reshape(1, H), 0.5 * W2.reshape(1, H),
         jnp.broadcast_to(b2.reshape(1, 1), (1, H))], axis=0)
    x0_out = _tc_copy(x0)
    xg, idxs, cnt16 = _sc_compact_gather(x1i, x0)
    h = _tc_mlp(cnt16, xg, wpack, x0_out)
    x2_new = _sc_scatter(idxs, cnt16, h.reshape(N), x2.reshape(N))
    return (x0_out, x1, x2_new.reshape(N, 1))


# speedup vs baseline: 28.6343x; 1.0698x over previous
import jax
import jax.numpy as jnp
from jax import lax
from jax.experimental import pallas as pl
from jax.experimental.pallas import tpu as pltpu
from jax.experimental.pallas import tpu_sc as plsc
import dataclasses
import functools

_sc_params = pltpu.CompilerParams()
if "needs_layout_passes" in pltpu.CompilerParams.__dataclass_fields__:
    _sc_params = dataclasses.replace(_sc_params, needs_layout_passes=False)

N = 524288
D = 128
H = 21
TASK = 3

NC = 2
NS = 16
NW = NC * NS
CHUNK = N // NW
KG = 128
BT = 2048
NBLK = N // BT
BPW = CHUNK // BT
BC = 8192

_mesh = plsc.VectorSubcoreMesh(core_axis_name="c", subcore_axis_name="s")


def _wid():
    return lax.axis_index("s") * NC + lax.axis_index("c")


@functools.partial(
    pl.kernel,
    out_type=[
        jax.ShapeDtypeStruct((N, D), jnp.float32),
        jax.ShapeDtypeStruct((NW, CHUNK), jnp.int32),
        jax.ShapeDtypeStruct((NW, 16), jnp.int32),
    ],
    mesh=_mesh,
    scratch_types=[
        pltpu.VMEM((CHUNK,), jnp.int32),
        pltpu.VMEM((CHUNK + KG,), jnp.int32),
        pltpu.VMEM((KG, D), jnp.float32),
        pltpu.VMEM((KG, D), jnp.float32),
        pltpu.VMEM((16,), jnp.int32),
        pltpu.SemaphoreType.DMA,
        pltpu.SemaphoreType.DMA,
        pltpu.SemaphoreType.DMA,
        pltpu.SemaphoreType.DMA,
    ],
    compiler_params=_sc_params,
)
def _sc_compact_gather(x1_hbm, x0_hbm, xg_hbm, idx_hbm, cnt_hbm,
                       x1_v, idx_v, rows_a, rows_b, cnt_v,
                       sem_ga, sem_gb, sem_wa, sem_wb):
    w = _wid()
    base = w * CHUNK
    pltpu.sync_copy(x1_hbm.at[pl.ds(base, CHUNK)], x1_v)

    lanes = lax.iota(jnp.int32, 16)

    @pl.loop(0, CHUNK // 16, init_carry=jnp.int32(0), unroll=4)
    def cnt(i, c):
        v = x1_v[pl.ds(i * 16, 16)]
        m = v == TASK
        gi = (base + i * 16) + lanes
        plsc.store_compressed(idx_v.at[pl.ds(c, 16)], gi, mask=m)
        return c + jnp.sum(m.astype(jnp.int32))

    safe = jnp.full((16,), base, jnp.int32)
    for p in range(KG // 16):
        idx_v[pl.ds(cnt + p * 16, 16)] = safe

    cnt_v[...] = jnp.full((16,), cnt, jnp.int32)
    pltpu.sync_copy(cnt_v, cnt_hbm.at[w])
    pltpu.sync_copy(idx_v.at[pl.ds(0, CHUNK)], idx_hbm.at[w])

    nch = (cnt + (KG - 1)) // KG

    def _start_gather(j, buf, sem):
        pltpu.make_async_copy(
            x0_hbm.at[idx_v.at[pl.ds(j * KG, KG)]], buf, sem
        ).start()

    def _wait_gather(buf, sem):
        pltpu.make_async_copy(x0_hbm.at[pl.ds(0, KG)], buf, sem).wait()

    def _start_write(j, buf, sem):
        pltpu.make_async_copy(
            buf, xg_hbm.at[pl.ds(base + j * KG, KG)], sem
        ).start()

    def _wait_write(buf, sem):
        pltpu.make_async_copy(buf, xg_hbm.at[pl.ds(0, KG)], sem).wait()

    @pl.when(nch > 0)
    def _():
        _start_gather(0, rows_a, sem_ga)

        @pl.loop(0, nch)
        def _(j):
            is_a = lax.rem(j, 2) == 0

            @pl.when(is_a)
            def _():
                _wait_gather(rows_a, sem_ga)

                @pl.when(j + 1 < nch)
                def _():
                    @pl.when(j >= 1)
                    def _():
                        _wait_write(rows_b, sem_wb)
                    _start_gather(j + 1, rows_b, sem_gb)
                _start_write(j, rows_a, sem_wa)

            @pl.when(jnp.logical_not(is_a))
            def _():
                _wait_gather(rows_b, sem_gb)

                @pl.when(j + 1 < nch)
                def _():
                    _wait_write(rows_a, sem_wa)
                    _start_gather(j + 1, rows_a, sem_ga)
                _start_write(j, rows_b, sem_wb)

        @pl.when(lax.rem(nch, 2) == 1)
        def _():
            _wait_write(rows_a, sem_wa)

            @pl.when(nch >= 2)
            def _():
                _wait_write(rows_b, sem_wb)

        @pl.when(lax.rem(nch, 2) == 0)
        def _():
            _wait_write(rows_b, sem_wb)
            _wait_write(rows_a, sem_wa)


def _copy_body(x_ref, o_ref):
    o_ref[...] = x_ref[...]


def _tc_copy(x0):
    return pl.pallas_call(
        _copy_body,
        grid=(N // BC,),
        in_specs=[pl.BlockSpec((BC, D), lambda i: (i, 0))],
        out_specs=pl.BlockSpec((BC, D), lambda i: (i, 0)),
        out_shape=jax.ShapeDtypeStruct((N, D), jnp.float32),
    )(x0)


def _mlp_body(cnt_ref, xg_ref, wp_ref, xo_any, h_ref):
    w = pl.program_id(0)
    j = pl.program_id(1)

    @pl.when(j * BT < cnt_ref[w, 0])
    def _():
        x = xg_ref[...]
        z = lax.dot_general(
            wp_ref[0:D, :], x, (((0,), (1,)), ((), ())),
            preferred_element_type=jnp.float32,
        )
        z = z + wp_ref[D:D + 1, :].reshape(H, 1)
        e = lax.erf(z * 0.7071067811865476)
        w2h = wp_ref[D + 1:D + 2, :].reshape(H, 1)
        b2 = wp_ref[D + 2, 0]
        h_ref[0] = jnp.sum((z + z * e) * w2h, axis=0, keepdims=True) + b2


def _xg_map(w, j, cnt_ref):
    nb = (cnt_ref[w, 0] + (BT - 1)) // BT
    jc = jnp.minimum(j, jnp.maximum(nb - 1, 0))
    return (w * BPW + jc, 0)


def _h_map(w, j, cnt_ref):
    b, _ = _xg_map(w, j, cnt_ref)
    return (b, 0, 0)


def _tc_mlp(cnt16, xg, wpack, x0_out, nbmax):
    return pl.pallas_call(
        _mlp_body,
        grid_spec=pltpu.PrefetchScalarGridSpec(
            num_scalar_prefetch=1,
            grid=(NW, nbmax),
            in_specs=[
                pl.BlockSpec((BT, D), _xg_map),
                pl.BlockSpec((D + 3, H), lambda w, j, c: (0, 0)),
                pl.BlockSpec(memory_space=pl.ANY),
            ],
            out_specs=pl.BlockSpec((1, 1, BT), _h_map),
        ),
        out_shape=jax.ShapeDtypeStruct((NBLK, 1, BT), jnp.float32),
    )(cnt16, xg, wpack, x0_out)


@functools.partial(
    pl.kernel,
    out_type=jax.ShapeDtypeStruct((N,), jnp.float32),
    mesh=_mesh,
    scratch_types=[
        pltpu.VMEM((CHUNK,), jnp.int32),
        pltpu.VMEM((CHUNK,), jnp.float32),
        pltpu.VMEM((CHUNK,), jnp.float32),
        pltpu.VMEM((16,), jnp.int32),
    ],
    compiler_params=_sc_params,
)
def _sc_scatter(idx_hbm, cnt_hbm, h_hbm, x2_hbm, out_hbm,
                idx_v, h_v, out_v, cnt_v):
    w = _wid()
    base = w * CHUNK
    pltpu.sync_copy(cnt_hbm.at[w], cnt_v)
    pltpu.sync_copy(idx_hbm.at[w], idx_v)
    pltpu.sync_copy(h_hbm.at[pl.ds(base, CHUNK)], h_v)
    pltpu.sync_copy(x2_hbm.at[pl.ds(base, CHUNK)], out_v)
    cnt = cnt_v[...][0]
    lanes = lax.iota(jnp.int32, 16)

    @pl.loop(0, (cnt + 15) // 16)
    def _(k):
        off = k * 16
        iv = idx_v[pl.ds(off, 16)] - base
        hv = h_v[pl.ds(off, 16)]
        m = (off + lanes) < cnt
        plsc.store_scatter(out_v, [iv], hv, mask=m)

    pltpu.sync_copy(out_v, out_hbm.at[pl.ds(base, CHUNK)])


def kernel(x0, x1, x2, W1, b1, W2, b2):
    x1i = x1.astype(jnp.int32).reshape(N)
    wpack = jnp.concatenate(
        [W1, b1.reshape(1, H), 0.5 * W2.reshape(1, H),
         jnp.broadcast_to(b2.reshape(1, 1), (1, H))], axis=0)
    x0_out = _tc_copy(x0)
    xg, idxs, cnt16 = _sc_compact_gather(x1i, x0)
    nbmax = jnp.max((cnt16[:, 0] + (BT - 1)) // BT)
    h = _tc_mlp(cnt16, xg, wpack, x0_out, nbmax)
    x2_new = _sc_scatter(idxs, cnt16, h.reshape(N), x2.reshape(N))
    return (x0_out, x1, x2_new.reshape(N, 1))


# speedup vs baseline: 31.3331x; 1.0942x over previous
import jax
import jax.numpy as jnp
from jax import lax
from jax.experimental import pallas as pl
from jax.experimental.pallas import tpu as pltpu
from jax.experimental.pallas import tpu_sc as plsc
import dataclasses
import functools

_sc_params = pltpu.CompilerParams()
if "needs_layout_passes" in pltpu.CompilerParams.__dataclass_fields__:
    _sc_params = dataclasses.replace(_sc_params, needs_layout_passes=False)

N = 524288
D = 128
H = 21
TASK = 3

NC = 2
NS = 16
NW = NC * NS
CHUNK = N // NW
KG = 128
BT = 2048
NBLK = N // BT
BPW = CHUNK // BT
BC = 8192

_mesh = plsc.VectorSubcoreMesh(core_axis_name="c", subcore_axis_name="s")


def _wid():
    return lax.axis_index("s") * NC + lax.axis_index("c")


@functools.partial(
    pl.kernel,
    out_type=[
        jax.ShapeDtypeStruct((N, D), jnp.float32),
        jax.ShapeDtypeStruct((NW, CHUNK), jnp.int32),
        jax.ShapeDtypeStruct((NW, 16), jnp.int32),
        jax.ShapeDtypeStruct((NW, 16), jnp.int32),
        jax.ShapeDtypeStruct((NC, 16), jnp.int32),
    ],
    mesh=_mesh,
    scratch_types=[
        pltpu.VMEM((CHUNK,), jnp.int32),
        pltpu.VMEM((CHUNK + KG,), jnp.int32),
        pltpu.VMEM((KG, D), jnp.float32),
        pltpu.VMEM((KG, D), jnp.float32),
        pltpu.VMEM((16,), jnp.int32),
        pltpu.VMEM((NS * 16,), jnp.int32),
        pltpu.VMEM_SHARED((NS * 16,), jnp.int32),
        pltpu.SemaphoreType.DMA,
        pltpu.SemaphoreType.DMA,
        pltpu.SemaphoreType.DMA,
        pltpu.SemaphoreType.DMA,
    ],
    compiler_params=_sc_params,
)
def _sc_compact_gather(x1_hbm, x0_hbm, xg_hbm, idx_hbm, cnt_hbm, off_hbm,
                       tot_hbm,
                       x1_v, idx_v, rows_a, rows_b, cnt_v, all_v, shared_v,
                       sem_ga, sem_gb, sem_wa, sem_wb):
    w = _wid()
    cidx = lax.axis_index("c")
    sid = lax.axis_index("s")
    base = w * CHUNK
    pltpu.sync_copy(x1_hbm.at[pl.ds(base, CHUNK)], x1_v)

    lanes = lax.iota(jnp.int32, 16)

    @pl.loop(0, CHUNK // 16, init_carry=jnp.int32(0), unroll=4)
    def cnt(i, c):
        v = x1_v[pl.ds(i * 16, 16)]
        m = v == TASK
        gi = (base + i * 16) + lanes
        plsc.store_compressed(idx_v.at[pl.ds(c, 16)], gi, mask=m)
        return c + jnp.sum(m.astype(jnp.int32))

    safe = jnp.full((16,), base, jnp.int32)
    for p in range(KG // 16):
        idx_v[pl.ds(cnt + p * 16, 16)] = safe

    cnt_v[...] = jnp.full((16,), cnt, jnp.int32)
    pltpu.sync_copy(cnt_v, cnt_hbm.at[w])
    pltpu.sync_copy(idx_v.at[pl.ds(0, CHUNK)], idx_hbm.at[w])

    nch = (cnt + (KG - 1)) // KG
    cntp = nch * KG

    cnt_v[...] = jnp.full((16,), cntp, jnp.int32)
    pltpu.sync_copy(cnt_v, shared_v.at[pl.ds(sid * 16, 16)])
    plsc.subcore_barrier()
    pltpu.sync_copy(shared_v, all_v)
    prefix = jnp.int32(0)
    tot = jnp.int32(0)
    for r in range(NS):
        cr = all_v[pl.ds(r * 16, 16)][0]
        prefix = prefix + jnp.where(r < sid, cr, 0)
        tot = tot + cr
    goff = pl.multiple_of(cidx * (N // NC) + prefix, KG)

    cnt_v[...] = jnp.full((16,), goff, jnp.int32)
    pltpu.sync_copy(cnt_v, off_hbm.at[w])

    @pl.when(sid == 0)
    def _():
        cnt_v[...] = jnp.full((16,), tot, jnp.int32)
        pltpu.sync_copy(cnt_v, tot_hbm.at[cidx])

    def _start_gather(j, buf, sem):
        pltpu.make_async_copy(
            x0_hbm.at[idx_v.at[pl.ds(j * KG, KG)]], buf, sem
        ).start()

    def _wait_gather(buf, sem):
        pltpu.make_async_copy(x0_hbm.at[pl.ds(0, KG)], buf, sem).wait()

    def _start_write(j, buf, sem):
        pltpu.make_async_copy(
            buf, xg_hbm.at[pl.ds(goff + j * KG, KG)], sem
        ).start()

    def _wait_write(buf, sem):
        pltpu.make_async_copy(buf, xg_hbm.at[pl.ds(0, KG)], sem).wait()

    @pl.when(nch > 0)
    def _():
        _start_gather(0, rows_a, sem_ga)

        @pl.loop(0, nch)
        def _(j):
            is_a = lax.rem(j, 2) == 0

            @pl.when(is_a)
            def _():
                _wait_gather(rows_a, sem_ga)

                @pl.when(j + 1 < nch)
                def _():
                    @pl.when(j >= 1)
                    def _():
                        _wait_write(rows_b, sem_wb)
                    _start_gather(j + 1, rows_b, sem_gb)
                _start_write(j, rows_a, sem_wa)

            @pl.when(jnp.logical_not(is_a))
            def _():
                _wait_gather(rows_b, sem_gb)

                @pl.when(j + 1 < nch)
                def _():
                    _wait_write(rows_a, sem_wa)
                    _start_gather(j + 1, rows_a, sem_ga)
                _start_write(j, rows_b, sem_wb)

        @pl.when(lax.rem(nch, 2) == 1)
        def _():
            _wait_write(rows_a, sem_wa)

            @pl.when(nch >= 2)
            def _():
                _wait_write(rows_b, sem_wb)

        @pl.when(lax.rem(nch, 2) == 0)
        def _():
            _wait_write(rows_b, sem_wb)
            _wait_write(rows_a, sem_wa)


def _copy_body(x_ref, o_ref):
    o_ref[...] = x_ref[...]


def _tc_copy(x0):
    return pl.pallas_call(
        _copy_body,
        grid=(N // BC,),
        in_specs=[pl.BlockSpec((BC, D), lambda i: (i, 0))],
        out_specs=pl.BlockSpec((BC, D), lambda i: (i, 0)),
        out_shape=jax.ShapeDtypeStruct((N, D), jnp.float32),
    )(x0)


def _mlp_body(tot_ref, xg_ref, wp_ref, xo_any, h_ref):
    c = pl.program_id(0)
    j = pl.program_id(1)

    @pl.when(j * BT < tot_ref[c, 0])
    def _():
        x = xg_ref[...]
        z = lax.dot_general(
            wp_ref[0:D, :], x, (((0,), (1,)), ((), ())),
            preferred_element_type=jnp.float32,
        )
        z = z + wp_ref[D:D + 1, :].reshape(H, 1)
        e = lax.erf(z * 0.7071067811865476)
        w2h = wp_ref[D + 1:D + 2, :].reshape(H, 1)
        b2 = wp_ref[D + 2, 0]
        h_ref[0] = jnp.sum((z + z * e) * w2h, axis=0, keepdims=True) + b2


BPC = N // NC // BT


def _xg_map(c, j, tot_ref):
    nb = (tot_ref[c, 0] + (BT - 1)) // BT
    jc = jnp.minimum(j, jnp.maximum(nb - 1, 0))
    return (c * BPC + jc, 0)


def _h_map(c, j, tot_ref):
    b, _ = _xg_map(c, j, tot_ref)
    return (b, 0, 0)


def _tc_mlp(tot, xg, wpack, x0_out, nbmax):
    return pl.pallas_call(
        _mlp_body,
        grid_spec=pltpu.PrefetchScalarGridSpec(
            num_scalar_prefetch=1,
            grid=(NC, nbmax),
            in_specs=[
                pl.BlockSpec((BT, D), _xg_map),
                pl.BlockSpec((D + 3, H), lambda c, j, t: (0, 0)),
                pl.BlockSpec(memory_space=pl.ANY),
            ],
            out_specs=pl.BlockSpec((1, 1, BT), _h_map),
        ),
        out_shape=jax.ShapeDtypeStruct((NBLK + BPW, 1, BT), jnp.float32),
    )(tot, xg, wpack, x0_out)


@functools.partial(
    pl.kernel,
    out_type=jax.ShapeDtypeStruct((N,), jnp.float32),
    mesh=_mesh,
    scratch_types=[
        pltpu.VMEM((CHUNK,), jnp.int32),
        pltpu.VMEM((CHUNK,), jnp.float32),
        pltpu.VMEM((CHUNK,), jnp.float32),
        pltpu.VMEM((16,), jnp.int32),
        pltpu.VMEM((16,), jnp.int32),
    ],
    compiler_params=_sc_params,
)
def _sc_scatter(idx_hbm, cnt_hbm, off_hbm, h_hbm, x2_hbm, out_hbm,
                idx_v, h_v, out_v, cnt_v, off_v):
    w = _wid()
    base = w * CHUNK
    pltpu.sync_copy(cnt_hbm.at[w], cnt_v)
    pltpu.sync_copy(off_hbm.at[w], off_v)
    pltpu.sync_copy(idx_hbm.at[w], idx_v)
    goff = pl.multiple_of(off_v[...][0], KG)
    pltpu.sync_copy(h_hbm.at[pl.ds(goff, CHUNK)], h_v)
    pltpu.sync_copy(x2_hbm.at[pl.ds(base, CHUNK)], out_v)
    cnt = cnt_v[...][0]
    lanes = lax.iota(jnp.int32, 16)

    @pl.loop(0, (cnt + 15) // 16)
    def _(k):
        off = k * 16
        iv = idx_v[pl.ds(off, 16)] - base
        hv = h_v[pl.ds(off, 16)]
        m = (off + lanes) < cnt
        plsc.store_scatter(out_v, [iv], hv, mask=m)

    pltpu.sync_copy(out_v, out_hbm.at[pl.ds(base, CHUNK)])


def kernel(x0, x1, x2, W1, b1, W2, b2):
    x1i = x1.astype(jnp.int32).reshape(N)
    wpack = jnp.concatenate(
        [W1, b1.reshape(1, H), 0.5 * W2.reshape(1, H),
         jnp.broadcast_to(b2.reshape(1, 1), (1, H))], axis=0)
    x0_out = _tc_copy(x0)
    xg, idxs, cnt16, off16, tot16 = _sc_compact_gather(x1i, x0)
    nbmax = jnp.max((tot16[:, 0] + (BT - 1)) // BT)
    h = _tc_mlp(tot16, xg, wpack, x0_out, nbmax)
    x2_new = _sc_scatter(idxs, cnt16, off16, h.reshape(N + CHUNK),
                         x2.reshape(N))
    return (x0_out, x1, x2_new.reshape(N, 1))


# speedup vs baseline: 31.3767x; 1.0014x over previous
import jax
import jax.numpy as jnp
from jax import lax
from jax.experimental import pallas as pl
from jax.experimental.pallas import tpu as pltpu
from jax.experimental.pallas import tpu_sc as plsc
import dataclasses
import functools

_sc_params = pltpu.CompilerParams()
if "needs_layout_passes" in pltpu.CompilerParams.__dataclass_fields__:
    _sc_params = dataclasses.replace(_sc_params, needs_layout_passes=False)

N = 524288
D = 128
H = 21
TASK = 3

NC = 2
NS = 16
NW = NC * NS
CHUNK = N // NW
KG = 128
BT = 2048
NBLK = N // BT
BPW = CHUNK // BT
BC = 8192

_mesh = plsc.VectorSubcoreMesh(core_axis_name="c", subcore_axis_name="s")


def _wid():
    return lax.axis_index("s") * NC + lax.axis_index("c")


@functools.partial(
    pl.kernel,
    out_type=[
        jax.ShapeDtypeStruct((N, D), jnp.float32),
        jax.ShapeDtypeStruct((NW, CHUNK), jnp.int32),
        jax.ShapeDtypeStruct((NW, 16), jnp.int32),
        jax.ShapeDtypeStruct((NW, 16), jnp.int32),
        jax.ShapeDtypeStruct((NC, 16), jnp.int32),
    ],
    mesh=_mesh,
    scratch_types=[
        pltpu.VMEM((CHUNK,), jnp.int32),
        pltpu.VMEM((CHUNK + KG,), jnp.int32),
        pltpu.VMEM((KG, D), jnp.float32),
        pltpu.VMEM((KG, D), jnp.float32),
        pltpu.VMEM((16,), jnp.int32),
        pltpu.VMEM((NS * 16,), jnp.int32),
        pltpu.VMEM_SHARED((NS * 16,), jnp.int32),
        pltpu.SemaphoreType.DMA,
        pltpu.SemaphoreType.DMA,
        pltpu.SemaphoreType.DMA,
        pltpu.SemaphoreType.DMA,
    ],
    compiler_params=_sc_params,
)
def _sc_compact_gather(x1_hbm, x0_hbm, xg_hbm, idx_hbm, cnt_hbm, off_hbm,
                       tot_hbm,
                       x1_v, idx_v, rows_a, rows_b, cnt_v, all_v, shared_v,
                       sem_ga, sem_gb, sem_wa, sem_wb):
    w = _wid()
    cidx = lax.axis_index("c")
    sid = lax.axis_index("s")
    base = w * CHUNK
    pltpu.sync_copy(x1_hbm.at[pl.ds(base, CHUNK)], x1_v)

    lanes = lax.iota(jnp.int32, 16)

    @pl.loop(0, CHUNK // 16, init_carry=jnp.int32(0), unroll=4)
    def cnt(i, c):
        v = x1_v[pl.ds(i * 16, 16)]
        m = v == TASK
        gi = (base + i * 16) + lanes
        plsc.store_compressed(idx_v.at[pl.ds(c, 16)], gi, mask=m)
        return c + jnp.sum(m.astype(jnp.int32))

    safe = jnp.full((16,), base, jnp.int32)
    for p in range(KG // 16):
        idx_v[pl.ds(cnt + p * 16, 16)] = safe

    cnt_v[...] = jnp.full((16,), cnt, jnp.int32)
    pltpu.sync_copy(cnt_v, cnt_hbm.at[w])
    pltpu.sync_copy(idx_v.at[pl.ds(0, CHUNK)], idx_hbm.at[w])

    nch = (cnt + (KG - 1)) // KG
    cntp = nch * KG

    cnt_v[...] = jnp.full((16,), cntp, jnp.int32)
    pltpu.sync_copy(cnt_v, shared_v.at[pl.ds(sid * 16, 16)])
    plsc.subcore_barrier()
    pltpu.sync_copy(shared_v, all_v)
    prefix = jnp.int32(0)
    tot = jnp.int32(0)
    for r in range(NS):
        cr = all_v[pl.ds(r * 16, 16)][0]
        prefix = prefix + jnp.where(r < sid, cr, 0)
        tot = tot + cr
    goff = pl.multiple_of(cidx * (N // NC) + prefix, KG)

    cnt_v[...] = jnp.full((16,), goff, jnp.int32)
    pltpu.sync_copy(cnt_v, off_hbm.at[w])

    @pl.when(sid == 0)
    def _():
        cnt_v[...] = jnp.full((16,), tot, jnp.int32)
        pltpu.sync_copy(cnt_v, tot_hbm.at[cidx])

    def _start_gather(j, buf, sem):
        pltpu.make_async_copy(
            x0_hbm.at[idx_v.at[pl.ds(j * KG, KG)]], buf, sem
        ).start()

    def _wait_gather(buf, sem):
        pltpu.make_async_copy(x0_hbm.at[pl.ds(0, KG)], buf, sem).wait()

    def _start_write(j, buf, sem):
        pltpu.make_async_copy(
            buf, xg_hbm.at[pl.ds(goff + j * KG, KG)], sem
        ).start()

    def _wait_write(buf, sem):
        pltpu.make_async_copy(buf, xg_hbm.at[pl.ds(0, KG)], sem).wait()

    @pl.when(nch > 0)
    def _():
        _start_gather(0, rows_a, sem_ga)

        @pl.loop(0, nch)
        def _(j):
            is_a = lax.rem(j, 2) == 0

            @pl.when(is_a)
            def _():
                _wait_gather(rows_a, sem_ga)

                @pl.when(j + 1 < nch)
                def _():
                    @pl.when(j >= 1)
                    def _():
                        _wait_write(rows_b, sem_wb)
                    _start_gather(j + 1, rows_b, sem_gb)
                _start_write(j, rows_a, sem_wa)

            @pl.when(jnp.logical_not(is_a))
            def _():
                _wait_gather(rows_b, sem_gb)

                @pl.when(j + 1 < nch)
                def _():
                    _wait_write(rows_a, sem_wa)
                    _start_gather(j + 1, rows_a, sem_ga)
                _start_write(j, rows_b, sem_wb)

        @pl.when(lax.rem(nch, 2) == 1)
        def _():
            _wait_write(rows_a, sem_wa)

            @pl.when(nch >= 2)
            def _():
                _wait_write(rows_b, sem_wb)

        @pl.when(lax.rem(nch, 2) == 0)
        def _():
            _wait_write(rows_b, sem_wb)
            _wait_write(rows_a, sem_wa)


def _copy_body(x_ref, o_ref):
    o_ref[...] = x_ref[...]


def _tc_copy(x0):
    return pl.pallas_call(
        _copy_body,
        grid=(N // BC,),
        in_specs=[pl.BlockSpec((BC, D), lambda i: (i, 0))],
        out_specs=pl.BlockSpec((BC, D), lambda i: (i, 0)),
        out_shape=jax.ShapeDtypeStruct((N, D), jnp.float32),
    )(x0)


def _mlp_body(tot_ref, xg_ref, wp_ref, xo_any, h_ref):
    c = pl.program_id(0)
    j = pl.program_id(1)

    @pl.when(j * BT < tot_ref[c, 0])
    def _():
        x = xg_ref[...]
        z = lax.dot_general(
            wp_ref[0:D, :], x, (((0,), (1,)), ((), ())),
            preferred_element_type=jnp.float32,
        )
        z = z + wp_ref[D:D + 1, :].reshape(H, 1)
        e = lax.erf(z * 0.7071067811865476)
        w2h = wp_ref[D + 1:D + 2, :].reshape(H, 1)
        b2 = wp_ref[D + 2, 0]
        h_ref[0] = jnp.sum((z + z * e) * w2h, axis=0, keepdims=True) + b2


BPC = N // NC // BT


def _xg_map(c, j, tot_ref):
    nb = (tot_ref[c, 0] + (BT - 1)) // BT
    jc = jnp.minimum(j, jnp.maximum(nb - 1, 0))
    return (c * BPC + jc, 0)


def _h_map(c, j, tot_ref):
    b, _ = _xg_map(c, j, tot_ref)
    return (b, 0, 0)


def _tc_mlp(tot, xg, wpack, x0_out, nbmax):
    return pl.pallas_call(
        _mlp_body,
        grid_spec=pltpu.PrefetchScalarGridSpec(
            num_scalar_prefetch=1,
            grid=(NC, nbmax),
            in_specs=[
                pl.BlockSpec((BT, D), _xg_map),
                pl.BlockSpec((D + 3, H), lambda c, j, t: (0, 0)),
                pl.BlockSpec(memory_space=pl.ANY),
            ],
            out_specs=pl.BlockSpec((1, 1, BT), _h_map),
        ),
        out_shape=jax.ShapeDtypeStruct((NBLK + BPW, 1, BT), jnp.float32),
    )(tot, xg, wpack, x0_out)


@functools.partial(
    pl.kernel,
    out_type=jax.ShapeDtypeStruct((N,), jnp.float32),
    mesh=_mesh,
    scratch_types=[
        pltpu.VMEM((CHUNK,), jnp.int32),
        pltpu.VMEM((CHUNK,), jnp.float32),
        pltpu.VMEM((CHUNK,), jnp.float32),
        pltpu.VMEM((16,), jnp.int32),
        pltpu.VMEM((16,), jnp.int32),
    ],
    compiler_params=_sc_params,
)
def _sc_scatter(idx_hbm, cnt_hbm, off_hbm, h_hbm, x2_hbm, out_hbm,
                idx_v, h_v, out_v, cnt_v, off_v):
    w = _wid()
    base = w * CHUNK
    pltpu.sync_copy(cnt_hbm.at[w], cnt_v)
    pltpu.sync_copy(off_hbm.at[w], off_v)
    pltpu.sync_copy(idx_hbm.at[w], idx_v)
    goff = pl.multiple_of(off_v[...][0], KG)
    pltpu.sync_copy(h_hbm.at[pl.ds(goff, CHUNK)], h_v)
    pltpu.sync_copy(x2_hbm.at[pl.ds(base, CHUNK)], out_v)
    cnt = cnt_v[...][0]
    lanes = lax.iota(jnp.int32, 16)

    @pl.loop(0, (cnt + 15) // 16)
    def _(k):
        off = k * 16
        iv = idx_v[pl.ds(off, 16)] - base
        hv = h_v[pl.ds(off, 16)]
        m = (off + lanes) < cnt
        plsc.store_scatter(out_v, [iv], hv, mask=m)

    pltpu.sync_copy(out_v, out_hbm.at[pl.ds(base, CHUNK)])


def kernel(x0, x1, x2, W1, b1, W2, b2):
    x1i = x1.astype(jnp.int32).reshape(N)
    wpack = jnp.concatenate(
        [W1, b1.reshape(1, H), 0.5 * W2.reshape(1, H),
         jnp.broadcast_to(b2.reshape(1, 1), (1, H))], axis=0)
    x0_out = _tc_copy(x0)
    xg, idxs, cnt16, off16, tot16 = _sc_compact_gather(x1i, x0)
    nbmax = jnp.maximum(jnp.max((tot16[:, 0] + (BT - 1)) // BT), 1)
    h = _tc_mlp(tot16, xg, wpack, x0_out, nbmax)
    x2_new = _sc_scatter(idxs, cnt16, off16, h.reshape(N + CHUNK),
                         x2.reshape(N))
    return (x0_out, x1, x2_new.reshape(N, 1))
